# initial kernel scaffold (unmeasured)
import jax
import jax.numpy as jnp
from jax import lax
from jax.experimental import pallas as pl
from jax.experimental.pallas import tpu as pltpu


def kernel(
    x,
):
    def body(*refs):
        pass

    out_shape = jax.ShapeDtypeStruct(..., jnp.float32)
    return pl.pallas_call(body, out_shape=out_shape)(...)


# baseline (device time: 602262 ns/iter reference)
import jax
import jax.numpy as jnp
from jax import lax
from jax.experimental import pallas as pl
from jax.experimental.pallas import tpu as pltpu

N_DEV = 4
M, N = 8192, 1024
HALF = M // 2
CHUNK = HALF // N_DEV
N_STEPS = 2 * (N_DEV - 1)


def kernel(x):
    def body(x_hbm, out_hbm, work, tmp_p, tmp_m,
             send_p, recv_p, send_m, recv_m, lcopy_sems):
        my_x = lax.axis_index("x")
        my_y = lax.axis_index("y")
        my_z = lax.axis_index("z")
        right = (my_y + 1) % N_DEV
        left = (my_y + N_DEV - 1) % N_DEV

        barrier = pltpu.get_barrier_semaphore()
        for nbr in (left, right):
            pl.semaphore_signal(
                barrier, inc=1,
                device_id=(my_x, nbr, my_z),
                device_id_type=pl.DeviceIdType.MESH,
            )
        pl.semaphore_wait(barrier, 2)

        in_copies = []
        for j in range(N_DEV):
            g_p = (my_y + j) % N_DEV
            g_m = (my_y + 2 * N_DEV - j) % N_DEV
            cp = pltpu.make_async_copy(
                x_hbm.at[pl.ds(g_p * CHUNK, CHUNK), :],
                work.at[pl.ds(j * CHUNK, CHUNK), :],
                lcopy_sems.at[2 * j],
            )
            cm = pltpu.make_async_copy(
                x_hbm.at[pl.ds(HALF + g_m * CHUNK, CHUNK), :],
                work.at[pl.ds(HALF + j * CHUNK, CHUNK), :],
                lcopy_sems.at[2 * j + 1],
            )
            cp.start()
            cm.start()
            in_copies += [cp, cm]
        for c in in_copies:
            c.wait()

        for s in range(N_STEPS):
            if s < N_DEV - 1:
                a = (-s) % N_DEV
                dst_p = tmp_p.at[s]
                dst_m = tmp_m.at[s]
            else:
                t = s - (N_DEV - 1)
                a = (1 - t) % N_DEV
                d = (a - 1) % N_DEV
                dst_p = work.at[pl.ds(d * CHUNK, CHUNK), :]
                dst_m = work.at[pl.ds(HALF + d * CHUNK, CHUNK), :]

            rdma_p = pltpu.make_async_remote_copy(
                src_ref=work.at[pl.ds(a * CHUNK, CHUNK), :],
                dst_ref=dst_p,
                send_sem=send_p.at[s],
                recv_sem=recv_p.at[s],
                device_id=(my_x, right, my_z),
                device_id_type=pl.DeviceIdType.MESH,
            )
            rdma_m = pltpu.make_async_remote_copy(
                src_ref=work.at[pl.ds(HALF + a * CHUNK, CHUNK), :],
                dst_ref=dst_m,
                send_sem=send_m.at[s],
                recv_sem=recv_m.at[s],
                device_id=(my_x, left, my_z),
                device_id_type=pl.DeviceIdType.MESH,
            )

            rdma_p.start()
            rdma_m.start()
            rdma_p.wait()
            rdma_m.wait()

            if s < N_DEV - 1:
                r = ((a - 1) % N_DEV) * CHUNK
                work[pl.ds(r, CHUNK), :] = (
                    work[pl.ds(r, CHUNK), :] + tmp_p[s, :, :]
                )
                rm = HALF + ((a - 1) % N_DEV) * CHUNK
                work[pl.ds(rm, CHUNK), :] = (
                    work[pl.ds(rm, CHUNK), :] + tmp_m[s, :, :]
                )

        out_copies = []
        for j in range(N_DEV):
            g_p = (my_y + j) % N_DEV
            g_m = (my_y + 2 * N_DEV - j) % N_DEV
            cp = pltpu.make_async_copy(
                work.at[pl.ds(j * CHUNK, CHUNK), :],
                out_hbm.at[pl.ds(g_p * CHUNK, CHUNK), :],
                lcopy_sems.at[2 * j],
            )
            cm = pltpu.make_async_copy(
                work.at[pl.ds(HALF + j * CHUNK, CHUNK), :],
                out_hbm.at[pl.ds(HALF + g_m * CHUNK, CHUNK), :],
                lcopy_sems.at[2 * j + 1],
            )
            cp.start()
            cm.start()
            out_copies += [cp, cm]
        for c in out_copies:
            c.wait()

    return pl.pallas_call(
        body,
        out_shape=jax.ShapeDtypeStruct((M, N), jnp.float32),
        in_specs=[pl.BlockSpec(memory_space=pl.ANY)],
        out_specs=pl.BlockSpec(memory_space=pl.ANY),
        scratch_shapes=[
            pltpu.VMEM((M, N), jnp.float32),
            pltpu.VMEM((N_DEV - 1, CHUNK, N), jnp.float32),
            pltpu.VMEM((N_DEV - 1, CHUNK, N), jnp.float32),
            pltpu.SemaphoreType.DMA((N_STEPS,)),
            pltpu.SemaphoreType.DMA((N_STEPS,)),
            pltpu.SemaphoreType.DMA((N_STEPS,)),
            pltpu.SemaphoreType.DMA((N_STEPS,)),
            pltpu.SemaphoreType.DMA((2 * N_DEV,)),
        ],
        compiler_params=pltpu.CompilerParams(
            collective_id=0,
            vmem_limit_bytes=60 * 1024 * 1024,
        ),
    )(x)


# device time: 294034 ns/iter; 2.0483x vs baseline; 2.0483x over previous
import jax
import jax.numpy as jnp
from jax import lax
from jax.experimental import pallas as pl
from jax.experimental.pallas import tpu as pltpu

N_DEV = 4
N_RING = 8
M, N = 8192, 1024
HALF = M // 2
CHUNK = HALF // N_DEV
CW = N // N_RING
P1_STEPS = 2 * (N_DEV - 1)
P2_STEPS = N_RING - 1
DO_PHASE2 = True


def kernel(x):
    def body(x_hbm, out_hbm, work, cbuf, t1p, t1m,
             s1p_snd, s1p_rcv, s1m_snd, s1m_rcv,
             s2a_snd, s2a_rcv, s2b_snd, s2b_rcv, lsems):
        my_x = lax.axis_index("x")
        my_y = lax.axis_index("y")
        my_z = lax.axis_index("z")
        yr = (my_y + 1) % N_DEV
        yl = (my_y + N_DEV - 1) % N_DEV
        p = jnp.where(my_x == 0, my_z, 2 * N_DEV - 1 - my_z)
        q = (p + 1) % N_RING
        r = (p + N_RING - 1) % N_RING
        qx = jnp.where(q < N_DEV, 0, 1)
        qz = jnp.where(q < N_DEV, q, 2 * N_DEV - 1 - q)
        rx = jnp.where(r < N_DEV, 0, 1)
        rz = jnp.where(r < N_DEV, r, 2 * N_DEV - 1 - r)

        barrier = pltpu.get_barrier_semaphore()
        for dev in ((my_x, yl, my_z), (my_x, yr, my_z),
                    (qx, my_y, qz), (rx, my_y, rz)):
            pl.semaphore_signal(
                barrier, inc=1, device_id=dev,
                device_id_type=pl.DeviceIdType.MESH,
            )
        pl.semaphore_wait(barrier, 4)

        col0 = p * CW
        init_copies = []
        for j in range(N_DEV):
            ga = (my_y + j) % N_DEV
            gb = (my_y + 2 * N_DEV - j) % N_DEV
            ca = pltpu.make_async_copy(
                x_hbm.at[pl.ds(ga * CHUNK, CHUNK), pl.ds(col0, CW)],
                cbuf.at[pl.ds(j * CHUNK, CHUNK), :],
                lsems.at[2 * j],
            )
            cb = pltpu.make_async_copy(
                x_hbm.at[pl.ds(HALF + gb * CHUNK, CHUNK), pl.ds(col0, CW)],
                cbuf.at[pl.ds(HALF + j * CHUNK, CHUNK), :],
                lsems.at[2 * j + 1],
            )
            ca.start()
            cb.start()
            init_copies += [ca, cb]
        for c in init_copies:
            c.wait()

        for s in range(P1_STEPS):
            if s < N_DEV - 1:
                a = (-s) % N_DEV
                dst_a = t1p.at[s]
                dst_b = t1m.at[s]
            else:
                t = s - (N_DEV - 1)
                a = (1 - t) % N_DEV
                d = (a - 1) % N_DEV
                dst_a = cbuf.at[pl.ds(d * CHUNK, CHUNK), :]
                dst_b = cbuf.at[pl.ds(HALF + d * CHUNK, CHUNK), :]

            rdma_a = pltpu.make_async_remote_copy(
                src_ref=cbuf.at[pl.ds(a * CHUNK, CHUNK), :],
                dst_ref=dst_a,
                send_sem=s1p_snd.at[s],
                recv_sem=s1p_rcv.at[s],
                device_id=(my_x, yr, my_z),
                device_id_type=pl.DeviceIdType.MESH,
            )
            rdma_b = pltpu.make_async_remote_copy(
                src_ref=cbuf.at[pl.ds(HALF + a * CHUNK, CHUNK), :],
                dst_ref=dst_b,
                send_sem=s1m_snd.at[s],
                recv_sem=s1m_rcv.at[s],
                device_id=(my_x, yl, my_z),
                device_id_type=pl.DeviceIdType.MESH,
            )
            rdma_a.start()
            rdma_b.start()
            rdma_a.wait()
            rdma_b.wait()

            if s < N_DEV - 1:
                d = (a - 1) % N_DEV
                cbuf[pl.ds(d * CHUNK, CHUNK), :] = (
                    cbuf[pl.ds(d * CHUNK, CHUNK), :] + t1p[s, :, :]
                )
                cbuf[pl.ds(HALF + d * CHUNK, CHUNK), :] = (
                    cbuf[pl.ds(HALF + d * CHUNK, CHUNK), :] + t1m[s, :, :]
                )

        work[0, :, :] = cbuf[:, :]

        for t in range(P2_STEPS if DO_PHASE2 else 0):
            st = (-t) % N_RING
            d = (st + N_RING - 1) % N_RING
            rdma_a = pltpu.make_async_remote_copy(
                src_ref=work.at[st, pl.ds(0, HALF), :],
                dst_ref=work.at[d, pl.ds(0, HALF), :],
                send_sem=s2a_snd.at[t],
                recv_sem=s2a_rcv.at[t],
                device_id=(qx, my_y, qz),
                device_id_type=pl.DeviceIdType.MESH,
            )
            rdma_b = pltpu.make_async_remote_copy(
                src_ref=work.at[st, pl.ds(HALF, HALF), :],
                dst_ref=work.at[d, pl.ds(HALF, HALF), :],
                send_sem=s2b_snd.at[t],
                recv_sem=s2b_rcv.at[t],
                device_id=(rx, my_y, rz),
                device_id_type=pl.DeviceIdType.MESH,
            )
            rdma_a.start()
            rdma_b.start()
            rdma_a.wait()
            rdma_b.wait()

        pend = []
        idx = 0
        for k in range(N_RING):
            gca = (p + k) % N_RING
            gcb = (p + N_RING - k) % N_RING
            for j in range(N_DEV):
                gra = (my_y + j) % N_DEV
                grb = (my_y + 2 * N_DEV - j) % N_DEV
                ca = pltpu.make_async_copy(
                    work.at[k, pl.ds(j * CHUNK, CHUNK), :],
                    out_hbm.at[pl.ds(gra * CHUNK, CHUNK),
                               pl.ds(gca * CW, CW)],
                    lsems.at[idx % 16],
                )
                ca.start()
                pend.append(ca)
                idx += 1
                cb = pltpu.make_async_copy(
                    work.at[k, pl.ds(HALF + j * CHUNK, CHUNK), :],
                    out_hbm.at[pl.ds(HALF + grb * CHUNK, CHUNK),
                               pl.ds(gcb * CW, CW)],
                    lsems.at[idx % 16],
                )
                cb.start()
                pend.append(cb)
                idx += 1
                if len(pend) == 16:
                    for c in pend:
                        c.wait()
                    pend = []
        for c in pend:
            c.wait()

    return pl.pallas_call(
        body,
        out_shape=jax.ShapeDtypeStruct((M, N), jnp.float32),
        in_specs=[pl.BlockSpec(memory_space=pl.ANY)],
        out_specs=pl.BlockSpec(memory_space=pl.ANY),
        scratch_shapes=[
            pltpu.VMEM((N_RING, M, CW), jnp.float32),
            pltpu.VMEM((M, CW), jnp.float32),
            pltpu.VMEM((N_DEV - 1, CHUNK, CW), jnp.float32),
            pltpu.VMEM((N_DEV - 1, CHUNK, CW), jnp.float32),
            pltpu.SemaphoreType.DMA((P1_STEPS,)),
            pltpu.SemaphoreType.DMA((P1_STEPS,)),
            pltpu.SemaphoreType.DMA((P1_STEPS,)),
            pltpu.SemaphoreType.DMA((P1_STEPS,)),
            pltpu.SemaphoreType.DMA((P2_STEPS,)),
            pltpu.SemaphoreType.DMA((P2_STEPS,)),
            pltpu.SemaphoreType.DMA((P2_STEPS,)),
            pltpu.SemaphoreType.DMA((P2_STEPS,)),
            pltpu.SemaphoreType.DMA((16,)),
        ],
        compiler_params=pltpu.CompilerParams(
            collective_id=0,
            vmem_limit_bytes=60 * 1024 * 1024,
        ),
    )(x)
